# baseline (speedup 1.0000x reference)
import functools

import jax
import jax.numpy as jnp
from jax import lax
from jax.experimental import pallas as pl
from jax.experimental.pallas import tpu as pltpu
from jax.experimental.pallas import tpu_sc as plsc

SEQ = 200
BATCH = 1024
VOCAB = 100000


def _make_kernel():
    nc, ns, nl = 2, 16, 16
    nw = nc * ns
    b_per_w = BATCH // nw

    mesh = plsc.VectorSubcoreMesh(core_axis_name="c", subcore_axis_name="s")

    @functools.partial(
        pl.kernel,
        mesh=mesh,
        out_type=jax.ShapeDtypeStruct((BATCH,), jnp.float32),
        compiler_params=pltpu.CompilerParams(
            needs_layout_passes=False, use_tc_tiling_on_sc=False
        ),
        scratch_types=[
            pltpu.VMEM((VOCAB,), jnp.float32),
            pltpu.VMEM((SEQ, b_per_w), jnp.int32),
            pltpu.VMEM((b_per_w,), jnp.float32),
            pltpu.VMEM((nl,), jnp.float32),
            pltpu.SemaphoreType.DMA,
            pltpu.SemaphoreType.DMA,
        ],
    )
    def k(text_hbm, w_hbm, b_hbm, out_hbm, w_v, tok_v, out_v, bias_v,
          sem_w, sem_t):
        wid = lax.axis_index("s") * nc + lax.axis_index("c")
        base = wid * b_per_w
        cp_w = pltpu.async_copy(w_hbm.at[0], w_v, sem_w)
        cp_t = pltpu.async_copy(text_hbm.at[:, pl.ds(base, b_per_w)], tok_v,
                                sem_t)
        pltpu.sync_copy(b_hbm, bias_v.at[pl.ds(0, 1)])
        bias = bias_v[...][0]
        with jax.named_scope("dma_wait"):
            cp_t.wait()
            cp_w.wait()

        def body(t, accs):
            a0, a1 = accs
            i0 = tok_v[t, pl.ds(0, nl)]
            i1 = tok_v[t, pl.ds(nl, nl)]
            return (a0 + plsc.load_gather(w_v, [i0]),
                    a1 + plsc.load_gather(w_v, [i1]))

        zero = jnp.zeros((nl,), jnp.float32)
        with jax.named_scope("gather_loop"):
            a0, a1 = lax.fori_loop(0, SEQ, body, (zero, zero), unroll=8)
        out_v[pl.ds(0, nl)] = a0 + bias
        out_v[pl.ds(nl, nl)] = a1 + bias
        pltpu.sync_copy(out_v, out_hbm.at[pl.ds(base, b_per_w)])

    return k


def kernel(text, W, b):
    out = _make_kernel()(text.astype(jnp.int32), W, b)
    return out.reshape(BATCH, 1)


# speedup vs baseline: 1.0901x; 1.0901x over previous
import functools

import jax
import jax.numpy as jnp
from jax import lax
from jax.experimental import pallas as pl
from jax.experimental.pallas import tpu as pltpu
from jax.experimental.pallas import tpu_sc as plsc

SEQ = 200
BATCH = 1024
VOCAB = 100000


def _make_kernel():
    nc, ns, nl = 2, 16, 16
    nw = nc * ns
    b_per_w = BATCH // nw

    mesh = plsc.VectorSubcoreMesh(core_axis_name="c", subcore_axis_name="s")

    @functools.partial(
        pl.kernel,
        mesh=mesh,
        out_type=jax.ShapeDtypeStruct((BATCH,), jnp.float32),
        compiler_params=pltpu.CompilerParams(
            needs_layout_passes=False, use_tc_tiling_on_sc=False
        ),
        scratch_types=[
            pltpu.VMEM((VOCAB,), jnp.float32),
            pltpu.VMEM((SEQ, b_per_w), jnp.int32),
            pltpu.VMEM((b_per_w,), jnp.float32),
            pltpu.VMEM((nl,), jnp.float32),
            pltpu.SemaphoreType.DMA,
            pltpu.SemaphoreType.DMA,
        ],
    )
    def k(text_hbm, w_hbm, b_hbm, out_hbm, w_v, tok_v, out_v, bias_v,
          sem_w, sem_t):
        sid = lax.axis_index("s")
        wid = sid * nc + lax.axis_index("c")
        base = wid * b_per_w
        chunk = 6248
        cps = []
        for j in range(16):
            c = lax.rem(sid + j, 16)
            cps.append(pltpu.async_copy(
                w_hbm.at[0, pl.ds(c * chunk, chunk)],
                w_v.at[pl.ds(c * chunk, chunk)], sem_w))
        cps.append(pltpu.async_copy(
            w_hbm.at[0, pl.ds(16 * chunk, VOCAB - 16 * chunk)],
            w_v.at[pl.ds(16 * chunk, VOCAB - 16 * chunk)], sem_w))
        cp_t = pltpu.async_copy(text_hbm.at[:, pl.ds(base, b_per_w)], tok_v,
                                sem_t)
        pltpu.sync_copy(b_hbm, bias_v.at[pl.ds(0, 1)])
        bias = bias_v[...][0]
        with jax.named_scope("dma_wait"):
            cp_t.wait()
            for cp in cps:
                cp.wait()

        def body(t, accs):
            a0, a1 = accs
            i0 = tok_v[t, pl.ds(0, nl)]
            i1 = tok_v[t, pl.ds(nl, nl)]
            return (a0 + plsc.load_gather(w_v, [i0]),
                    a1 + plsc.load_gather(w_v, [i1]))

        zero = jnp.zeros((nl,), jnp.float32)
        with jax.named_scope("gather_loop"):
            a0, a1 = lax.fori_loop(0, SEQ, body, (zero, zero), unroll=8)
        out_v[pl.ds(0, nl)] = a0 + bias
        out_v[pl.ds(nl, nl)] = a1 + bias
        pltpu.sync_copy(out_v, out_hbm.at[pl.ds(base, b_per_w)])

    return k


def kernel(text, W, b):
    out = _make_kernel()(text.astype(jnp.int32), W, b)
    return out.reshape(BATCH, 1)


# speedup vs baseline: 1.1994x; 1.1002x over previous
import functools

import jax
import jax.numpy as jnp
from jax import lax
from jax.experimental import pallas as pl
from jax.experimental.pallas import tpu as pltpu
from jax.experimental.pallas import tpu_sc as plsc

SEQ = 200
BATCH = 1024
VOCAB = 100000
HALF = VOCAB // 2


def _make_kernel():
    nc, ns, nl = 2, 16, 16
    b_per_pair = BATCH // (nc * ns // 2)
    groups = b_per_pair // nl

    mesh = plsc.VectorSubcoreMesh(core_axis_name="c", subcore_axis_name="s")

    @functools.partial(
        pl.kernel,
        mesh=mesh,
        out_type=jax.ShapeDtypeStruct((BATCH,), jnp.float32),
        compiler_params=pltpu.CompilerParams(
            needs_layout_passes=False, use_tc_tiling_on_sc=False
        ),
        scratch_types=[
            pltpu.VMEM((HALF,), jnp.float32),
            pltpu.VMEM((SEQ, b_per_pair), jnp.int32),
            pltpu.VMEM((b_per_pair,), jnp.float32),
            pltpu.VMEM((b_per_pair,), jnp.float32),
            pltpu.VMEM((nl,), jnp.float32),
            pltpu.VMEM_SHARED((BATCH,), jnp.float32),
            pltpu.SemaphoreType.DMA,
            pltpu.SemaphoreType.DMA,
        ],
    )
    def k(text_hbm, w_hbm, b_hbm, out_hbm, w_v, tok_v, part_v, peer_v,
          bias_v, shared, sem_w, sem_t):
        cid = lax.axis_index("c")
        sid = lax.axis_index("s")
        pair = sid // 2
        member = sid % 2
        blk = cid * (ns // 2) + pair
        lo = member * HALF

        chunk = 6248
        cps = []
        for j in range(8):
            c = lax.rem(sid + j, 8)
            cps.append(pltpu.async_copy(
                w_hbm.at[0, pl.ds(lo + c * chunk, chunk)],
                w_v.at[pl.ds(c * chunk, chunk)], sem_w))
        cps.append(pltpu.async_copy(
            w_hbm.at[0, pl.ds(lo + 8 * chunk, HALF - 8 * chunk)],
            w_v.at[pl.ds(8 * chunk, HALF - 8 * chunk)], sem_w))
        cp_t = pltpu.async_copy(
            text_hbm.at[:, pl.ds(blk * b_per_pair, b_per_pair)], tok_v, sem_t)
        pltpu.sync_copy(b_hbm, bias_v.at[pl.ds(0, 1)])
        bias = bias_v[...][0]
        with jax.named_scope("dma_wait"):
            cp_t.wait()
            for cp in cps:
                cp.wait()

        lo_v = jnp.full((nl,), lo, jnp.int32)
        hi_m1 = jnp.full((nl,), HALF - 1, jnp.int32)
        zero_i = jnp.zeros((nl,), jnp.int32)
        zero_f = jnp.zeros((nl,), jnp.float32)

        def body(t, accs):
            new = []
            for g in range(groups):
                idx = tok_v[t, pl.ds(g * nl, nl)]
                loc = idx - lo_v
                inb = (loc >= zero_i) & (loc <= hi_m1)
                loc_c = jnp.minimum(jnp.maximum(loc, zero_i), hi_m1)
                val = plsc.load_gather(w_v, [loc_c])
                new.append(accs[g] + jnp.where(inb, val, zero_f))
            return tuple(new)

        with jax.named_scope("gather_loop"):
            accs = lax.fori_loop(0, SEQ, body, (zero_f,) * groups, unroll=4)
        for g in range(groups):
            part_v[pl.ds(g * nl, nl)] = accs[g]

        slot = (pair * 2 + member) * b_per_pair
        pltpu.sync_copy(part_v, shared.at[pl.ds(slot, b_per_pair)])
        plsc.subcore_barrier()

        @pl.when(member == 0)
        def _():
            pltpu.sync_copy(
                shared.at[pl.ds(slot + b_per_pair, b_per_pair)], peer_v)
            for g in range(groups):
                s = pl.ds(g * nl, nl)
                part_v[s] = part_v[s] + peer_v[s] + bias
            pltpu.sync_copy(
                part_v, out_hbm.at[pl.ds(blk * b_per_pair, b_per_pair)])

    return k


def kernel(text, W, b):
    out = _make_kernel()(text.astype(jnp.int32), W, b)
    return out.reshape(BATCH, 1)
